# baseline (speedup 1.0000x reference)
import functools

import jax
import jax.numpy as jnp
from jax import lax
from jax.experimental import pallas as pl
from jax.experimental.pallas import tpu as pltpu
from jax.experimental.pallas import tpu_sc as plsc

C = 1.0
EPS = 1e-15
PROJ_EPS = 4e-3
N_NODES = 10000
D = 128
E = 320000

NC = 2
NS = 16
NW = NC * NS
EPT = E // NW
CH = 40
EPS2 = E // NS
NCH2 = EPS2 // CH
IBLK = 50
NIB = NCH2 // IBLK
CHD = 100
IBLKD = 20
NIBD = EPS2 // (CHD * IBLKD)
NP = 10240
RPT = NP // NS

def _deg_kernel_body(dst_hbm, zrows_hbm, ones_hbm, degp_hbm, dst_v, ones_v,
                     deg_sh):
    s = lax.axis_index("s")
    pltpu.sync_copy(zrows_hbm, deg_sh.at[pl.ds(s * RPT, RPT)])
    pltpu.sync_copy(ones_hbm, ones_v)
    plsc.subcore_barrier()

    def iblock(ib, carry):
        pltpu.sync_copy(dst_hbm.at[s].at[ib], dst_v)

        def chunk(j, carry2):
            pltpu.sync_copy(ones_v, deg_sh.at[dst_v.at[j]], add=True)
            return carry2

        lax.fori_loop(0, IBLKD, chunk, 0)
        return carry

    lax.fori_loop(0, NIBD, iblock, 0)
    plsc.subcore_barrier()
    pltpu.sync_copy(
        deg_sh.at[pl.ds(s * RPT, RPT)],
        degp_hbm.at[pl.ds(s * RPT, RPT)],
    )


def _agg_kernel_body(y_hbm, src_hbm, dst_hbm, zrows_hbm, s_hbm,
                     src_v, dst_v, rows_v, agg_sh, gsem0, gsem1,
                     ssem0, ssem1):
    s = lax.axis_index("s")
    pltpu.sync_copy(zrows_hbm, agg_sh.at[pl.ds(s * RPT, RPT)])
    plsc.subcore_barrier()

    def iblock(ib, carry):
        pltpu.sync_copy(src_hbm.at[s].at[ib], src_v)
        pltpu.sync_copy(dst_hbm.at[s].at[ib], dst_v)
        pltpu.make_async_copy(
            y_hbm.at[src_v.at[0]], rows_v.at[0], gsem0).start()
        pltpu.make_async_copy(
            y_hbm.at[src_v.at[1]], rows_v.at[1], gsem1).start()

        def pair(i, carry2):
            j = 2 * i
            pltpu.make_async_copy(
                y_hbm.at[src_v.at[j]], rows_v.at[0], gsem0).wait()
            d0 = pltpu.async_copy(
                rows_v.at[0], agg_sh.at[dst_v.at[j]], ssem0, add=True)
            pltpu.make_async_copy(
                y_hbm.at[src_v.at[j + 1]], rows_v.at[1], gsem1).wait()
            d1 = pltpu.async_copy(
                rows_v.at[1], agg_sh.at[dst_v.at[j + 1]], ssem1, add=True)
            d0.wait()
            pltpu.make_async_copy(
                y_hbm.at[src_v.at[j + 2]], rows_v.at[0], gsem0).start()
            d1.wait()
            pltpu.make_async_copy(
                y_hbm.at[src_v.at[j + 3]], rows_v.at[1], gsem1).start()
            return carry2

        lax.fori_loop(0, IBLK // 2 - 1, pair, 0)
        pltpu.make_async_copy(
            y_hbm.at[src_v.at[IBLK - 2]], rows_v.at[0], gsem0).wait()
        pltpu.sync_copy(rows_v.at[0], agg_sh.at[dst_v.at[IBLK - 2]],
                        add=True)
        pltpu.make_async_copy(
            y_hbm.at[src_v.at[IBLK - 1]], rows_v.at[1], gsem1).wait()
        pltpu.sync_copy(rows_v.at[1], agg_sh.at[dst_v.at[IBLK - 1]],
                        add=True)
        return carry

    lax.fori_loop(0, NIB, iblock, 0)
    plsc.subcore_barrier()
    pltpu.sync_copy(
        agg_sh.at[pl.ds(s * RPT, RPT)],
        s_hbm.at[pl.ds(s * RPT, RPT)],
    )


def _norm(v):
    return jnp.sqrt(jnp.sum(v * v, axis=-1, keepdims=True))


def _artanh(x):
    x = jnp.clip(x, -1.0 + 1e-7, 1.0 - 1e-7)
    return 0.5 * jnp.log((1.0 + x) / (1.0 - x))


def _proj(x):
    norm = jnp.clip(_norm(x), EPS, None)
    maxnorm = 1.0 - PROJ_EPS
    return jnp.where(norm > maxnorm, x / norm * maxnorm, x)


def _expmap0(u):
    un = jnp.clip(_norm(u), EPS, None)
    return jnp.tanh(un) * u / un


def _logmap0(p):
    pn = jnp.clip(_norm(p), EPS, None)
    return p / pn * _artanh(pn)


def _mobius_add(x, y):
    x2 = jnp.sum(x * x, axis=-1, keepdims=True)
    y2 = jnp.sum(y * y, axis=-1, keepdims=True)
    xy = jnp.sum(x * y, axis=-1, keepdims=True)
    num = (1.0 + 2.0 * xy + y2) * x + (1.0 - x2) * y
    denom = 1.0 + 2.0 * xy + x2 * y2
    return num / jnp.clip(denom, EPS, None)


def _dinv_from_deg(degp):
    return lax.rsqrt(degp[:, 0:1] + 1.0)


BLK = 1000


def _tc_dense_body(x_ref, wt_ref, b_ref, degp_ref, y_ref):
    x = x_ref[...]
    wt = wt_ref[...]
    b = b_ref[...]
    xh = _proj(_expmap0(x))
    xn = jnp.clip(_norm(xh), EPS, None)
    mx = jnp.dot(xh, wt, preferred_element_type=jnp.float32)
    mxn = jnp.clip(_norm(mx), EPS, None)
    res = jnp.tanh(mxn / xn * _artanh(xn)) * mx / mxn
    zero_mask = jnp.all(mx == 0, axis=-1, keepdims=True)
    res = jnp.where(zero_mask, jnp.zeros_like(res), res)
    res = _proj(res)
    hyp_b = _proj(_expmap0(b))
    res = _proj(_mobius_add(res, hyp_b))
    xt = _logmap0(res)
    dinv = _dinv_from_deg(degp_ref[...])
    y_ref[...] = xt * dinv


def _tc_final_body(s_ref, degp_ref, out_ref):
    dinv = _dinv_from_deg(degp_ref[...])
    agg = s_ref[...] * dinv
    out = _proj(_expmap0(agg))
    out_t = jax.nn.relu(_logmap0(out))
    out_ref[...] = _proj(_expmap0(out_t))


_tc_dense = pl.pallas_call(
    _tc_dense_body,
    out_shape=jax.ShapeDtypeStruct((N_NODES, D), jnp.float32),
    grid=(N_NODES // BLK,),
    in_specs=[
        pl.BlockSpec((BLK, D), lambda i: (i, 0)),
        pl.BlockSpec((D, D), lambda i: (0, 0)),
        pl.BlockSpec((1, D), lambda i: (0, 0)),
        pl.BlockSpec((BLK, D), lambda i: (i, 0)),
    ],
    out_specs=pl.BlockSpec((BLK, D), lambda i: (i, 0)),
)

_tc_final = pl.pallas_call(
    _tc_final_body,
    out_shape=jax.ShapeDtypeStruct((N_NODES, D), jnp.float32),
    grid=(N_NODES // BLK,),
    in_specs=[
        pl.BlockSpec((BLK, D), lambda i: (i, 0)),
        pl.BlockSpec((BLK, D), lambda i: (i, 0)),
    ],
    out_specs=pl.BlockSpec((BLK, D), lambda i: (i, 0)),
)


@functools.cache
def _sc_kernels():
    mesh = plsc.VectorSubcoreMesh(
        core_axis_name="c", subcore_axis_name="s",
        num_cores=NC, num_subcores=NS)
    mesh1 = plsc.VectorSubcoreMesh(
        core_axis_name="c", subcore_axis_name="s",
        num_cores=1, num_subcores=NS)
    deg_kernel = pl.kernel(
        _deg_kernel_body,
        mesh=mesh1,
        out_type=jax.ShapeDtypeStruct((NP, D), jnp.float32),
        scratch_types=[
            pltpu.VMEM((IBLKD, CHD), jnp.int32),
            pltpu.VMEM((CHD, D), jnp.float32),
            pltpu.VMEM_SHARED((NP, D), jnp.float32),
        ],
    )
    agg_kernel = pl.kernel(
        _agg_kernel_body,
        mesh=mesh1,
        out_type=jax.ShapeDtypeStruct((NP, D), jnp.float32),
        scratch_types=[
            pltpu.VMEM((IBLK, CH), jnp.int32),
            pltpu.VMEM((IBLK, CH), jnp.int32),
            pltpu.VMEM((2, CH, D), jnp.float32),
            pltpu.VMEM_SHARED((NP, D), jnp.float32),
            pltpu.SemaphoreType.DMA,
            pltpu.SemaphoreType.DMA,
            pltpu.SemaphoreType.DMA,
            pltpu.SemaphoreType.DMA,
        ],
    )
    return deg_kernel, agg_kernel


def kernel(x, edge_index, W, b):
    deg_kernel, agg_kernel = _sc_kernels()
    src = edge_index[0].astype(jnp.int32)
    dst = edge_index[1].astype(jnp.int32)
    src_agg = src.reshape(NS, NIB, IBLK, CH)
    dst_agg = dst.reshape(NS, NIB, IBLK, CH)
    dst_deg = dst.reshape(NS, NIBD, IBLKD, CHD)
    ones_rows = jnp.ones((CHD, D), jnp.float32)
    zrows = jnp.zeros((RPT, D), jnp.float32)

    degp = deg_kernel(dst_deg, zrows, ones_rows)
    y = _tc_dense(x, W.T, b.reshape(1, D), degp)
    s = agg_kernel(y, src_agg, dst_agg, zrows)
    return _tc_final(s, degp)


# speedup vs baseline: 1.2071x; 1.2071x over previous
import functools

import jax
import jax.numpy as jnp
from jax import lax
from jax.experimental import pallas as pl
from jax.experimental.pallas import tpu as pltpu
from jax.experimental.pallas import tpu_sc as plsc

C = 1.0
EPS = 1e-15
PROJ_EPS = 4e-3
N_NODES = 10000
D = 128
E = 320000

NC = 2
NS = 16
NW = NC * NS
EPT = E // NW
CH = 80
EPS2 = E // NS
NCH2 = EPS2 // CH
IBLK = 10
NIB = NCH2 // IBLK
CHD = 100
IBLKD = 20
NIBD = EPS2 // (CHD * IBLKD)
NP = 10240
RPT = NP // NS

def _deg_kernel_body(dst_hbm, zrows_hbm, ones_hbm, degp_hbm, dst_v, ones_v,
                     deg_sh):
    s = lax.axis_index("s")
    pltpu.sync_copy(zrows_hbm, deg_sh.at[pl.ds(s * RPT, RPT)])
    pltpu.sync_copy(ones_hbm, ones_v)
    plsc.subcore_barrier()

    def iblock(ib, carry):
        pltpu.sync_copy(dst_hbm.at[s].at[ib], dst_v)

        def chunk(j, carry2):
            pltpu.sync_copy(ones_v, deg_sh.at[dst_v.at[j]], add=True)
            return carry2

        lax.fori_loop(0, IBLKD, chunk, 0)
        return carry

    lax.fori_loop(0, NIBD, iblock, 0)
    plsc.subcore_barrier()
    pltpu.sync_copy(
        deg_sh.at[pl.ds(s * RPT, RPT)],
        degp_hbm.at[pl.ds(s * RPT, RPT)],
    )


def _agg_kernel_body(y_hbm, src_hbm, dst_hbm, zrows_hbm, s_hbm,
                     src_v, dst_v, rows_v, agg_sh, gsem0, gsem1,
                     ssem0, ssem1):
    s = lax.axis_index("s")
    pltpu.sync_copy(zrows_hbm, agg_sh.at[pl.ds(s * RPT, RPT)])
    plsc.subcore_barrier()

    def iblock(ib, carry):
        pltpu.sync_copy(src_hbm.at[s].at[ib], src_v)
        pltpu.sync_copy(dst_hbm.at[s].at[ib], dst_v)
        pltpu.make_async_copy(
            y_hbm.at[src_v.at[0]], rows_v.at[0], gsem0).start()

        def pair(i, carry2):
            j = 2 * i
            pltpu.make_async_copy(
                y_hbm.at[src_v.at[j + 1]], rows_v.at[1], gsem1).start()
            pltpu.make_async_copy(
                y_hbm.at[src_v.at[j]], rows_v.at[0], gsem0).wait()
            pltpu.sync_copy(rows_v.at[0], agg_sh.at[dst_v.at[j]], add=True)

            @pl.when(j + 2 < IBLK)
            def _():
                pltpu.make_async_copy(
                    y_hbm.at[src_v.at[j + 2]], rows_v.at[0], gsem0).start()

            pltpu.make_async_copy(
                y_hbm.at[src_v.at[j + 1]], rows_v.at[1], gsem1).wait()
            pltpu.sync_copy(rows_v.at[1], agg_sh.at[dst_v.at[j + 1]],
                            add=True)
            return carry2

        lax.fori_loop(0, IBLK // 2, pair, 0)
        return carry

    lax.fori_loop(0, NIB, iblock, 0)
    plsc.subcore_barrier()
    pltpu.sync_copy(
        agg_sh.at[pl.ds(s * RPT, RPT)],
        s_hbm.at[pl.ds(s * RPT, RPT)],
    )


def _norm(v):
    return jnp.sqrt(jnp.sum(v * v, axis=-1, keepdims=True))


def _artanh(x):
    x = jnp.clip(x, -1.0 + 1e-7, 1.0 - 1e-7)
    return 0.5 * jnp.log((1.0 + x) / (1.0 - x))


def _proj(x):
    norm = jnp.clip(_norm(x), EPS, None)
    maxnorm = 1.0 - PROJ_EPS
    return jnp.where(norm > maxnorm, x / norm * maxnorm, x)


def _expmap0(u):
    un = jnp.clip(_norm(u), EPS, None)
    return jnp.tanh(un) * u / un


def _logmap0(p):
    pn = jnp.clip(_norm(p), EPS, None)
    return p / pn * _artanh(pn)


def _mobius_add(x, y):
    x2 = jnp.sum(x * x, axis=-1, keepdims=True)
    y2 = jnp.sum(y * y, axis=-1, keepdims=True)
    xy = jnp.sum(x * y, axis=-1, keepdims=True)
    num = (1.0 + 2.0 * xy + y2) * x + (1.0 - x2) * y
    denom = 1.0 + 2.0 * xy + x2 * y2
    return num / jnp.clip(denom, EPS, None)


def _dinv_from_deg(degp):
    return lax.rsqrt(degp[:, 0:1] + 1.0)


BLK = 1000


def _tc_dense_body(x_ref, wt_ref, b_ref, degp_ref, y_ref):
    x = x_ref[...]
    wt = wt_ref[...]
    b = b_ref[...]
    xh = _proj(_expmap0(x))
    xn = jnp.clip(_norm(xh), EPS, None)
    mx = jnp.dot(xh, wt, preferred_element_type=jnp.float32)
    mxn = jnp.clip(_norm(mx), EPS, None)
    res = jnp.tanh(mxn / xn * _artanh(xn)) * mx / mxn
    zero_mask = jnp.all(mx == 0, axis=-1, keepdims=True)
    res = jnp.where(zero_mask, jnp.zeros_like(res), res)
    res = _proj(res)
    hyp_b = _proj(_expmap0(b))
    res = _proj(_mobius_add(res, hyp_b))
    xt = _logmap0(res)
    dinv = _dinv_from_deg(degp_ref[...])
    y_ref[...] = xt * dinv


def _tc_final_body(s_ref, degp_ref, out_ref):
    dinv = _dinv_from_deg(degp_ref[...])
    agg = s_ref[...] * dinv
    out = _proj(_expmap0(agg))
    out_t = jax.nn.relu(_logmap0(out))
    out_ref[...] = _proj(_expmap0(out_t))


_tc_dense = pl.pallas_call(
    _tc_dense_body,
    out_shape=jax.ShapeDtypeStruct((N_NODES, D), jnp.float32),
    grid=(N_NODES // BLK,),
    in_specs=[
        pl.BlockSpec((BLK, D), lambda i: (i, 0)),
        pl.BlockSpec((D, D), lambda i: (0, 0)),
        pl.BlockSpec((1, D), lambda i: (0, 0)),
        pl.BlockSpec((BLK, D), lambda i: (i, 0)),
    ],
    out_specs=pl.BlockSpec((BLK, D), lambda i: (i, 0)),
)

_tc_final = pl.pallas_call(
    _tc_final_body,
    out_shape=jax.ShapeDtypeStruct((N_NODES, D), jnp.float32),
    grid=(N_NODES // BLK,),
    in_specs=[
        pl.BlockSpec((BLK, D), lambda i: (i, 0)),
        pl.BlockSpec((BLK, D), lambda i: (i, 0)),
    ],
    out_specs=pl.BlockSpec((BLK, D), lambda i: (i, 0)),
)


@functools.cache
def _sc_kernels():
    mesh = plsc.VectorSubcoreMesh(
        core_axis_name="c", subcore_axis_name="s",
        num_cores=NC, num_subcores=NS)
    mesh1 = plsc.VectorSubcoreMesh(
        core_axis_name="c", subcore_axis_name="s",
        num_cores=1, num_subcores=NS)
    deg_kernel = pl.kernel(
        _deg_kernel_body,
        mesh=mesh1,
        out_type=jax.ShapeDtypeStruct((NP, D), jnp.float32),
        scratch_types=[
            pltpu.VMEM((IBLKD, CHD), jnp.int32),
            pltpu.VMEM((CHD, D), jnp.float32),
            pltpu.VMEM_SHARED((NP, D), jnp.float32),
        ],
    )
    agg_kernel = pl.kernel(
        _agg_kernel_body,
        mesh=mesh1,
        out_type=jax.ShapeDtypeStruct((NP, D), jnp.float32),
        scratch_types=[
            pltpu.VMEM((IBLK, CH), jnp.int32),
            pltpu.VMEM((IBLK, CH), jnp.int32),
            pltpu.VMEM((2, CH, D), jnp.float32),
            pltpu.VMEM_SHARED((NP, D), jnp.float32),
            pltpu.SemaphoreType.DMA,
            pltpu.SemaphoreType.DMA,
            pltpu.SemaphoreType.DMA,
            pltpu.SemaphoreType.DMA,
        ],
    )
    return deg_kernel, agg_kernel


def kernel(x, edge_index, W, b):
    deg_kernel, agg_kernel = _sc_kernels()
    src = edge_index[0].astype(jnp.int32)
    dst = edge_index[1].astype(jnp.int32)
    src_agg = src.reshape(NS, NIB, IBLK, CH)
    dst_agg = dst.reshape(NS, NIB, IBLK, CH)
    dst_deg = dst.reshape(NS, NIBD, IBLKD, CHD)
    ones_rows = jnp.ones((CHD, D), jnp.float32)
    zrows = jnp.zeros((RPT, D), jnp.float32)

    degp = deg_kernel(dst_deg, zrows, ones_rows)
    y = _tc_dense(x, W.T, b.reshape(1, D), degp)
    s = agg_kernel(y, src_agg, dst_agg, zrows)
    return _tc_final(s, degp)
